# baseline (speedup 1.0000x reference)
import functools

import jax
import jax.numpy as jnp
from jax import lax
from jax.experimental import pallas as pl
from jax.experimental.pallas import tpu as pltpu
from jax.experimental.pallas import tpu_sc as plsc

NC, NS, L = 2, 16, 16
NW = NC * NS
BATCH, SEQ, D = 16384, 50, 32
K = 512
NBC = BATCH // K
UNITS = SEQ * NBC
PER_W = UNITS // NW
GROUPS = K // L

MAX_NORM = 1.0
EPS = 1e-7


def _renorm_unit(rows_v, outb, lane):
    @pl.loop(0, GROUPS)
    def _grp(g):
        ridx = g * L + lane
        cols = []
        ss = jnp.zeros((L,), jnp.float32)
        for j in range(D):
            fidx = (lane + j) & (D - 1)
            v = plsc.load_gather(rows_v, [ridx, fidx])
            cols.append(v)
            ss = ss + v * v
        bits = plsc.bitcast(ss, jnp.int32)
        y = plsc.bitcast(
            jnp.int32(0x5F3759DF) - lax.shift_right_logical(bits, 1), jnp.float32)
        for _ in range(3):
            y = y * (1.5 - 0.5 * ss * y * y)
        norm = ss * y
        scale = jnp.where(ss > MAX_NORM * MAX_NORM, MAX_NORM / (norm + EPS), 1.0)
        for j in range(D):
            fidx = (lane + j) & (D - 1)
            plsc.store_scatter(outb, [fidx, ridx], cols[j] * scale)


def _sc_body(idxt_hbm, table_hbm, out_hbm,
             idx0, idx1, rows0, rows1, outb0, outb1,
             gsem0, gsem1, osem0, osem1):
    wid = lax.axis_index("s") * NC + lax.axis_index("c")
    lane = lax.iota(jnp.int32, L)
    idxv = (idx0, idx1)
    rows = (rows0, rows1)
    outb = (outb0, outb1)
    gsem = (gsem0, gsem1)
    osem = (osem0, osem1)

    def coords(k):
        u = wid + k * NW
        return u // NBC, (u % NBC) * K

    def issue_gather(k, b):
        s, b0 = coords(k)
        pltpu.sync_copy(idxt_hbm.at[s, pl.ds(b0, K)], idxv[b])
        pltpu.async_copy(table_hbm.at[idxv[b]], rows[b], gsem[b])

    def wait_gather(b):
        pltpu.make_async_copy(table_hbm.at[idxv[b]], rows[b], gsem[b]).wait()

    def issue_out(k, b):
        s, b0 = coords(k)
        pltpu.async_copy(outb[b], out_hbm.at[s, :, pl.ds(b0, K)], osem[b])

    def wait_out(b):
        pltpu.make_async_copy(
            outb[b], out_hbm.at[0, :, pl.ds(0, K)], osem[b]).wait()

    issue_gather(0, 0)

    @pl.loop(0, PER_W // 2)
    def _pair(m):
        kA = 2 * m
        issue_gather(kA + 1, 1)
        wait_gather(0)

        @pl.when(m > 0)
        def _():
            wait_out(0)

        _renorm_unit(rows[0], outb[0], lane)
        issue_out(kA, 0)

        @pl.when(m + 1 < PER_W // 2)
        def _():
            issue_gather(kA + 2, 0)

        wait_gather(1)

        @pl.when(m > 0)
        def _():
            wait_out(1)

        _renorm_unit(rows[1], outb[1], lane)
        issue_out(kA + 1, 1)

    wait_out(0)
    wait_out(1)


@jax.jit
def _lookup_renorm(idxt, table):
    mesh = plsc.VectorSubcoreMesh(core_axis_name="c", subcore_axis_name="s")
    return pl.kernel(
        _sc_body,
        out_type=jax.ShapeDtypeStruct((SEQ, D, BATCH), jnp.float32),
        mesh=mesh,
        scratch_types=[
            pltpu.VMEM((K,), jnp.int32),
            pltpu.VMEM((K,), jnp.int32),
            pltpu.VMEM((K, D), jnp.float32),
            pltpu.VMEM((K, D), jnp.float32),
            pltpu.VMEM((D, K), jnp.float32),
            pltpu.VMEM((D, K), jnp.float32),
            pltpu.SemaphoreType.DMA,
            pltpu.SemaphoreType.DMA,
            pltpu.SemaphoreType.DMA,
            pltpu.SemaphoreType.DMA,
        ],
        compiler_params=pltpu.CompilerParams(
            needs_layout_passes=False, use_tc_tiling_on_sc=False),
    )(idxt, table)


def kernel(indices, table):
    idxt = indices.T.astype(jnp.int32)
    out = _lookup_renorm(idxt, table)
    return out.transpose(2, 0, 1)


# speedup vs baseline: 1.0533x; 1.0533x over previous
import functools

import jax
import jax.numpy as jnp
from jax import lax
from jax.experimental import pallas as pl
from jax.experimental.pallas import tpu as pltpu
from jax.experimental.pallas import tpu_sc as plsc

NC, NS, L = 2, 16, 16
NW = NC * NS
BATCH, SEQ, D = 16384, 50, 32
K = 512
NBC = BATCH // K
UNITS = SEQ * NBC
PER_W = UNITS // NW
GROUPS = K // L

MAX_NORM = 1.0
EPS = 1e-7


def _renorm_unit(rows_v, outb, lane):
    @pl.loop(0, GROUPS)
    def _grp(g):
        ridx = g * L + lane
        cols = []
        acc = [jnp.zeros((L,), jnp.float32) for _ in range(4)]
        for j in range(D):
            fidx = (lane + j) & (D - 1)
            v = plsc.load_gather(rows_v, [ridx, fidx])
            cols.append(v)
            acc[j % 4] = acc[j % 4] + v * v
        ss = (acc[0] + acc[1]) + (acc[2] + acc[3])
        bits = plsc.bitcast(ss, jnp.int32)
        y = plsc.bitcast(
            jnp.int32(0x5F3759DF) - lax.shift_right_logical(bits, 1), jnp.float32)
        for _ in range(2):
            y = y * (1.5 - 0.5 * ss * y * y)
        norm = ss * y
        scale = jnp.where(ss > MAX_NORM * MAX_NORM, MAX_NORM / (norm + EPS), 1.0)
        for j in range(D):
            fidx = (lane + j) & (D - 1)
            plsc.store_scatter(outb, [fidx, ridx], cols[j] * scale)


def _sc_body(idxt_hbm, table_hbm, out_hbm,
             idx0, idx1, rows0, rows1, outb0, outb1,
             gsem0, gsem1, osem0, osem1):
    wid = lax.axis_index("s") * NC + lax.axis_index("c")
    lane = lax.iota(jnp.int32, L)
    idxv = (idx0, idx1)
    rows = (rows0, rows1)
    outb = (outb0, outb1)
    gsem = (gsem0, gsem1)
    osem = (osem0, osem1)

    def coords(k):
        u = wid + k * NW
        return u // NBC, (u % NBC) * K

    def issue_gather(k, b):
        s, b0 = coords(k)
        pltpu.sync_copy(idxt_hbm.at[s, pl.ds(b0, K)], idxv[b])
        pltpu.async_copy(table_hbm.at[idxv[b]], rows[b], gsem[b])

    def wait_gather(b):
        pltpu.make_async_copy(table_hbm.at[idxv[b]], rows[b], gsem[b]).wait()

    def issue_out(k, b):
        s, b0 = coords(k)
        pltpu.async_copy(outb[b], out_hbm.at[s, :, pl.ds(b0, K)], osem[b])

    def wait_out(b):
        pltpu.make_async_copy(
            outb[b], out_hbm.at[0, :, pl.ds(0, K)], osem[b]).wait()

    issue_gather(0, 0)

    @pl.loop(0, PER_W // 2)
    def _pair(m):
        kA = 2 * m
        issue_gather(kA + 1, 1)
        wait_gather(0)

        @pl.when(m > 0)
        def _():
            wait_out(0)

        _renorm_unit(rows[0], outb[0], lane)
        issue_out(kA, 0)

        @pl.when(m + 1 < PER_W // 2)
        def _():
            issue_gather(kA + 2, 0)

        wait_gather(1)

        @pl.when(m > 0)
        def _():
            wait_out(1)

        _renorm_unit(rows[1], outb[1], lane)
        issue_out(kA + 1, 1)

    wait_out(0)
    wait_out(1)


@jax.jit
def _lookup_renorm(idxt, table):
    mesh = plsc.VectorSubcoreMesh(core_axis_name="c", subcore_axis_name="s")
    return pl.kernel(
        _sc_body,
        out_type=jax.ShapeDtypeStruct((SEQ, D, BATCH), jnp.float32),
        mesh=mesh,
        scratch_types=[
            pltpu.VMEM((K,), jnp.int32),
            pltpu.VMEM((K,), jnp.int32),
            pltpu.VMEM((K, D), jnp.float32),
            pltpu.VMEM((K, D), jnp.float32),
            pltpu.VMEM((D, K), jnp.float32),
            pltpu.VMEM((D, K), jnp.float32),
            pltpu.SemaphoreType.DMA,
            pltpu.SemaphoreType.DMA,
            pltpu.SemaphoreType.DMA,
            pltpu.SemaphoreType.DMA,
        ],
        compiler_params=pltpu.CompilerParams(
            needs_layout_passes=False, use_tc_tiling_on_sc=False),
    )(idxt, table)


def kernel(indices, table):
    idxt = indices.T.astype(jnp.int32)
    out = _lookup_renorm(idxt, table)
    return out.transpose(2, 0, 1)


# speedup vs baseline: 1.0916x; 1.0364x over previous
import functools

import jax
import jax.numpy as jnp
from jax import lax
from jax.experimental import pallas as pl
from jax.experimental.pallas import tpu as pltpu
from jax.experimental.pallas import tpu_sc as plsc

NC, NS, L = 2, 16, 16
NW = NC * NS
BATCH, SEQ, D = 16384, 50, 32
K = 512
NBC = BATCH // K
UNITS = SEQ * NBC
PER_W = UNITS // NW
GROUPS = K // L

MAX_NORM = 1.0
EPS = 1e-7


def _renorm_unit(rows_v, outb, lane):
    @pl.loop(0, GROUPS)
    def _grp(g):
        ridx = g * L + lane
        cols = []
        acc = [jnp.zeros((L,), jnp.float32) for _ in range(4)]
        for j in range(D):
            fidx = (lane + j) & (D - 1)
            v = plsc.load_gather(rows_v, [ridx, fidx])
            cols.append(v)
            acc[j % 4] = acc[j % 4] + v * v
        ss = (acc[0] + acc[1]) + (acc[2] + acc[3])
        bits = plsc.bitcast(ss, jnp.int32)
        y = plsc.bitcast(
            jnp.int32(0x5F3759DF) - lax.shift_right_logical(bits, 1), jnp.float32)
        for _ in range(2):
            y = y * (1.5 - 0.5 * ss * y * y)
        norm = ss * y
        scale = jnp.where(ss > MAX_NORM * MAX_NORM, MAX_NORM / (norm + EPS), 1.0)
        for j in range(D):
            fidx = (lane + j) & (D - 1)
            plsc.store_scatter(outb, [fidx, ridx], cols[j] * scale)


def _sc_body(idxt_hbm, table_hbm, out_hbm,
             idxall, rows0, rows1, rows2, rows3, outb0, outb1,
             isem, gsem0, gsem1, gsem2, gsem3, osem0, osem1):
    wid = lax.axis_index("s") * NC + lax.axis_index("c")
    lane = lax.iota(jnp.int32, L)
    rows = (rows0, rows1, rows2, rows3)
    outb = (outb0, outb1)
    gsem = (gsem0, gsem1, gsem2, gsem3)
    osem = (osem0, osem1)

    def coords(k):
        u = wid + k * NW
        return u // NBC, (u % NBC) * K

    for k in range(PER_W):
        s, b0 = coords(k)
        pltpu.async_copy(idxt_hbm.at[s, pl.ds(b0, K)], idxall.at[k], isem)
    pltpu.make_async_copy(
        idxt_hbm.at[pl.ds(0, PER_W), pl.ds(0, K)], idxall, isem).wait()

    def issue_gather(k):
        b = k % 4
        pltpu.async_copy(table_hbm.at[idxall.at[k]], rows[b], gsem[b])

    def issue_gather_dyn(k, b):
        pltpu.async_copy(table_hbm.at[idxall.at[k]], rows[b], gsem[b])

    def wait_gather(b):
        pltpu.make_async_copy(
            table_hbm.at[pl.ds(0, K)], rows[b], gsem[b]).wait()

    def issue_out(k, b):
        s, b0 = coords(k)
        pltpu.async_copy(outb[b], out_hbm.at[s, :, pl.ds(b0, K)], osem[b])

    def wait_out(b):
        pltpu.make_async_copy(
            outb[b], out_hbm.at[0, :, pl.ds(0, K)], osem[b]).wait()

    for k in range(4):
        issue_gather(k)

    @pl.loop(0, PER_W // 4)
    def _quad(m):
        k0 = 4 * m
        for off in range(4):
            k = k0 + off
            ob = off % 2
            wait_gather(off)
            if off < 2:
                @pl.when(m > 0)
                def _():
                    wait_out(ob)
            else:
                wait_out(ob)
            _renorm_unit(rows[off], outb[ob], lane)
            issue_out(k, ob)

            @pl.when(k + 4 < PER_W)
            def _():
                issue_gather_dyn(k + 4, off)

    for k in (PER_W - 2, PER_W - 1):
        off = k % 4
        ob = k % 2
        wait_gather(off)
        wait_out(ob)
        _renorm_unit(rows[off], outb[ob], lane)
        issue_out(k, ob)

    wait_out(0)
    wait_out(1)


@jax.jit
def _lookup_renorm(idxt, table):
    mesh = plsc.VectorSubcoreMesh(core_axis_name="c", subcore_axis_name="s")
    return pl.kernel(
        _sc_body,
        out_type=jax.ShapeDtypeStruct((SEQ, D, BATCH), jnp.float32),
        mesh=mesh,
        scratch_types=[
            pltpu.VMEM((PER_W, K), jnp.int32),
            pltpu.VMEM((K, D), jnp.float32),
            pltpu.VMEM((K, D), jnp.float32),
            pltpu.VMEM((K, D), jnp.float32),
            pltpu.VMEM((K, D), jnp.float32),
            pltpu.VMEM((D, K), jnp.float32),
            pltpu.VMEM((D, K), jnp.float32),
            pltpu.SemaphoreType.DMA,
            pltpu.SemaphoreType.DMA,
            pltpu.SemaphoreType.DMA,
            pltpu.SemaphoreType.DMA,
            pltpu.SemaphoreType.DMA,
            pltpu.SemaphoreType.DMA,
            pltpu.SemaphoreType.DMA,
        ],
        compiler_params=pltpu.CompilerParams(
            needs_layout_passes=False, use_tc_tiling_on_sc=False),
    )(idxt, table)


def kernel(indices, table):
    idxt = indices.T.astype(jnp.int32)
    out = _lookup_renorm(idxt, table)
    return out.transpose(2, 0, 1)
